# baseline (speedup 1.0000x reference)
import functools

import jax
import jax.numpy as jnp
from jax import lax
from jax.experimental import pallas as pl
from jax.experimental.pallas import tpu as pltpu
from jax.experimental.pallas import tpu_sc as plsc

NC = 2
NS = 16
GW = 125
NG = 2
CHUNK = GW * NG
SUP = 2000
ZB = 16


def _sc_aggregate(x2, col2, row2, vals, *, n, d, e):
    ep = e // NS
    nsup = ep // SUP
    supc = SUP // CHUNK
    stripe = n // NS // 8 * 8
    last_stripe = n - stripe * (NS - 1)

    mesh = plsc.VectorSubcoreMesh(core_axis_name="c", subcore_axis_name="s")

    @functools.partial(
        pl.kernel,
        out_type=jax.ShapeDtypeStruct((NC * n, d), jnp.float32),
        mesh=mesh,
        scratch_types=[
            pltpu.VMEM((SUP // GW, GW), jnp.int32),
            pltpu.VMEM((SUP // GW, GW), jnp.int32),
            pltpu.VMEM((SUP,), jnp.float32),
            pltpu.VMEM((CHUNK, d), jnp.float32),
            pltpu.VMEM_SHARED((n, d), jnp.float32),
            pltpu.SemaphoreType.DMA,
        ],
    )
    def body(x_hbm, col_hbm, row_hbm, val_hbm, out_hbm,
             colv, rowv, valv, rows_v, agg, sem):
        c = lax.axis_index("c")
        s = lax.axis_index("s")

        def zfill(r, carry):
            for u in range(d // 16):
                rows_v[r, pl.ds(u * 16, 16)] = jnp.zeros((16,), jnp.float32)
            return carry
        lax.fori_loop(0, ZB, zfill, 0)
        sbase = pl.multiple_of(s * stripe, 8)
        nblk = jnp.where(s == NS - 1, last_stripe // ZB, stripe // ZB)

        def zcopy(t, carry):
            off = pl.multiple_of(sbase + t * ZB, 8)
            pltpu.sync_copy(rows_v.at[pl.ds(0, ZB)], agg.at[pl.ds(off, ZB)])
            return carry
        lax.fori_loop(0, nblk, zcopy, 0)
        plsc.subcore_barrier()

        def sup_body(k, carry):
            ebase = c * e + s * ep + k * SUP
            ibase = pl.multiple_of(ebase // GW, 8)
            pltpu.sync_copy(col_hbm.at[pl.ds(ibase, SUP // GW)], colv)
            pltpu.sync_copy(row_hbm.at[pl.ds(ibase, SUP // GW)], rowv)
            pltpu.sync_copy(val_hbm.at[pl.ds(ebase, SUP)], valv)

            def chunk_body(i, ccarry):
                descs = [
                    pltpu.async_copy(
                        x_hbm.at[colv.at[i * NG + j]],
                        rows_v.at[pl.ds(j * GW, GW)],
                        sem,
                    )
                    for j in range(NG)
                ]
                for dsc in descs:
                    dsc.wait()

                @plsc.parallel_loop(0, CHUNK, unroll=2)
                def edge_body(ei):
                    base16 = ei // 16 * 16
                    grp = valv[pl.ds(i * CHUNK + base16, 16)]
                    v16 = grp.at[jnp.full((16,), ei - base16, jnp.int32)].get(
                        mode="promise_in_bounds")
                    for u in range(d // 16):
                        sl = (ei, pl.ds(u * 16, 16))
                        rows_v[sl] = rows_v[sl] * v16

                for j in range(NG):
                    pltpu.sync_copy(
                        rows_v.at[pl.ds(j * GW, GW)],
                        agg.at[rowv.at[i * NG + j]],
                        add=True,
                    )
                return ccarry
            lax.fori_loop(0, supc, chunk_body, 0)
            return carry
        lax.fori_loop(0, nsup, sup_body, 0)
        plsc.subcore_barrier()

        def drain(t, carry):
            off = pl.multiple_of(sbase + t * ZB, 8)
            pltpu.sync_copy(
                agg.at[pl.ds(off, ZB)],
                out_hbm.at[pl.ds(pl.multiple_of(c * n + sbase + t * ZB, 8), ZB)],
            )
            return carry
        lax.fori_loop(0, nblk, drain, 0)

    return body(x2, col2, row2, vals)


def _mm_relu_kernel(a_ref, w_ref, o_ref):
    o_ref[...] = jnp.maximum(
        jnp.dot(a_ref[...], w_ref[...], preferred_element_type=jnp.float32),
        0.0,
    )


def kernel(x, adj_indices, adj_values, W):
    b, n, d = x.shape
    e = adj_indices.shape[1]
    dout = W.shape[1]

    row = adj_indices[..., 0].astype(jnp.int32)
    col = adj_indices[..., 1].astype(jnp.int32)
    colg = col + (jnp.arange(b, dtype=jnp.int32) * n)[:, None]
    col2 = colg.reshape(b * e // GW, GW)
    row2 = row.reshape(b * e // GW, GW)
    vals = adj_values.reshape(b * e)
    x2 = x.reshape(b * n, d)

    agg = _sc_aggregate(x2, col2, row2, vals, n=n, d=d, e=e)

    rows_total = b * n
    blk = 2000
    out = pl.pallas_call(
        _mm_relu_kernel,
        grid=(rows_total // blk,),
        in_specs=[
            pl.BlockSpec((blk, d), lambda i: (i, 0)),
            pl.BlockSpec((d, dout), lambda i: (0, 0)),
        ],
        out_specs=pl.BlockSpec((blk, dout), lambda i: (i, 0)),
        out_shape=jax.ShapeDtypeStruct((rows_total, dout), jnp.float32),
    )(agg, W)
    return out.reshape(b, n, dout)


# speedup vs baseline: 1.0034x; 1.0034x over previous
import functools

import jax
import jax.numpy as jnp
from jax import lax
from jax.experimental import pallas as pl
from jax.experimental.pallas import tpu as pltpu
from jax.experimental.pallas import tpu_sc as plsc

NC = 2
NS = 16
GW = 125
NG = 2
CHUNK = GW * NG
SUP = 2000
ZB = 16


def _sc_aggregate(x2, col2, row2, vals, *, n, d, e):
    ep = e // NS
    nsup = ep // SUP
    supc = SUP // CHUNK
    stripe = n // NS // 8 * 8
    last_stripe = n - stripe * (NS - 1)

    mesh = plsc.VectorSubcoreMesh(core_axis_name="c", subcore_axis_name="s")

    @functools.partial(
        pl.kernel,
        out_type=jax.ShapeDtypeStruct((NC * n, d), jnp.float32),
        mesh=mesh,
        scratch_types=[
            pltpu.VMEM((SUP // GW, GW), jnp.int32),
            pltpu.VMEM((SUP // GW, GW), jnp.int32),
            pltpu.VMEM((SUP,), jnp.float32),
            pltpu.VMEM((CHUNK, d), jnp.float32),
            pltpu.VMEM_SHARED((n, d), jnp.float32),
            pltpu.SemaphoreType.DMA,
        ],
    )
    def body(x_hbm, col_hbm, row_hbm, val_hbm, out_hbm,
             colv, rowv, valv, rows_v, agg, sem):
        c = lax.axis_index("c")
        s = lax.axis_index("s")

        def zfill(r, carry):
            for u in range(d // 16):
                rows_v[r, pl.ds(u * 16, 16)] = jnp.zeros((16,), jnp.float32)
            return carry
        lax.fori_loop(0, ZB, zfill, 0)
        sbase = pl.multiple_of(s * stripe, 8)
        nblk = jnp.where(s == NS - 1, last_stripe // ZB, stripe // ZB)

        def zcopy(t, carry):
            off = pl.multiple_of(sbase + t * ZB, 8)
            pltpu.sync_copy(rows_v.at[pl.ds(0, ZB)], agg.at[pl.ds(off, ZB)])
            return carry
        lax.fori_loop(0, nblk, zcopy, 0)
        plsc.subcore_barrier()

        def sup_body(k, carry):
            ebase = c * e + s * ep + k * SUP
            ibase = pl.multiple_of(ebase // GW, 8)
            pltpu.sync_copy(col_hbm.at[pl.ds(ibase, SUP // GW)], colv)
            pltpu.sync_copy(row_hbm.at[pl.ds(ibase, SUP // GW)], rowv)
            pltpu.sync_copy(val_hbm.at[pl.ds(ebase, SUP)], valv)

            def chunk_body(i, ccarry):
                descs = [
                    pltpu.async_copy(
                        x_hbm.at[colv.at[i * NG + j]],
                        rows_v.at[pl.ds(j * GW, GW)],
                        sem,
                    )
                    for j in range(NG)
                ]
                for dsc in descs:
                    dsc.wait()

                @plsc.parallel_loop(0, CHUNK, unroll=4)
                def edge_body(ei):
                    base16 = ei // 16 * 16
                    grp = valv[pl.ds(i * CHUNK + base16, 16)]
                    v16 = grp.at[jnp.full((16,), ei - base16, jnp.int32)].get(
                        mode="promise_in_bounds")
                    for u in range(d // 16):
                        sl = (ei, pl.ds(u * 16, 16))
                        rows_v[sl] = rows_v[sl] * v16

                for j in range(NG):
                    pltpu.sync_copy(
                        rows_v.at[pl.ds(j * GW, GW)],
                        agg.at[rowv.at[i * NG + j]],
                        add=True,
                    )
                return ccarry
            lax.fori_loop(0, supc, chunk_body, 0)
            return carry
        lax.fori_loop(0, nsup, sup_body, 0)
        plsc.subcore_barrier()

        def drain(t, carry):
            off = pl.multiple_of(sbase + t * ZB, 8)
            pltpu.sync_copy(
                agg.at[pl.ds(off, ZB)],
                out_hbm.at[pl.ds(pl.multiple_of(c * n + sbase + t * ZB, 8), ZB)],
            )
            return carry
        lax.fori_loop(0, nblk, drain, 0)

    return body(x2, col2, row2, vals)


def _mm_relu_kernel(a_ref, w_ref, o_ref):
    o_ref[...] = jnp.maximum(
        jnp.dot(a_ref[...], w_ref[...], preferred_element_type=jnp.float32),
        0.0,
    )


def kernel(x, adj_indices, adj_values, W):
    b, n, d = x.shape
    e = adj_indices.shape[1]
    dout = W.shape[1]

    row = adj_indices[..., 0].astype(jnp.int32)
    col = adj_indices[..., 1].astype(jnp.int32)
    colg = col + (jnp.arange(b, dtype=jnp.int32) * n)[:, None]
    col2 = colg.reshape(b * e // GW, GW)
    row2 = row.reshape(b * e // GW, GW)
    vals = adj_values.reshape(b * e)
    x2 = x.reshape(b * n, d)

    agg = _sc_aggregate(x2, col2, row2, vals, n=n, d=d, e=e)

    rows_total = b * n
    blk = 2000
    out = pl.pallas_call(
        _mm_relu_kernel,
        grid=(rows_total // blk,),
        in_specs=[
            pl.BlockSpec((blk, d), lambda i: (i, 0)),
            pl.BlockSpec((d, dout), lambda i: (0, 0)),
        ],
        out_specs=pl.BlockSpec((blk, dout), lambda i: (i, 0)),
        out_shape=jax.ShapeDtypeStruct((rows_total, dout), jnp.float32),
    )(agg, W)
    return out.reshape(b, n, dout)


# speedup vs baseline: 1.3394x; 1.3348x over previous
import functools

import jax
import jax.numpy as jnp
from jax import lax
from jax.experimental import pallas as pl
from jax.experimental.pallas import tpu as pltpu
from jax.experimental.pallas import tpu_sc as plsc

NC = 2
NS = 16
GW = 125
SUP = 2000
ZB = 16


def _sc_aggregate(x2, col2, row2, vals, *, n, d, e):
    ep = e // NS
    nch = ep // GW
    supc = SUP // GW
    nsup = ep // SUP
    stripe = n // NS // 8 * 8
    last_stripe = n - stripe * (NS - 1)

    mesh = plsc.VectorSubcoreMesh(core_axis_name="c", subcore_axis_name="s")

    @functools.partial(
        pl.kernel,
        out_type=jax.ShapeDtypeStruct((NC * n, d), jnp.float32),
        mesh=mesh,
        scratch_types=[
            pltpu.VMEM((ep // GW, GW), jnp.int32),
            pltpu.VMEM((2, supc, GW), jnp.int32),
            pltpu.VMEM((SUP,), jnp.float32),
            pltpu.VMEM((2, GW, d), jnp.float32),
            pltpu.VMEM_SHARED((n, d), jnp.float32),
            pltpu.SemaphoreType.DMA,
            pltpu.SemaphoreType.DMA,
            pltpu.SemaphoreType.DMA,
        ],
    )
    def body(x_hbm, col_hbm, row_hbm, val_hbm, out_hbm,
             colv, rowv, valv, bufs, agg, gsem, ssem, stsem):
        c = lax.axis_index("c")
        s = lax.axis_index("s")

        ebase = c * e + s * ep
        cb = pl.multiple_of(ebase // GW, 8)

        pltpu.sync_copy(col_hbm.at[pl.ds(cb, nch)], colv)
        pltpu.sync_copy(row_hbm.at[pl.ds(cb, supc)], rowv.at[0])
        pltpu.sync_copy(val_hbm.at[pl.ds(ebase, SUP)], valv)

        def bfill(r, carry):
            for u in range(d // 16):
                bufs[1, r, pl.ds(u * 16, 16)] = jnp.zeros((16,), jnp.float32)
            return carry
        lax.fori_loop(0, GW, bfill, 0)

        sbase = pl.multiple_of(s * stripe, 8)
        nblk = jnp.where(s == NS - 1, last_stripe // ZB, stripe // ZB)

        def zcopy(t, carry):
            off = pl.multiple_of(sbase + t * ZB, 8)
            pltpu.sync_copy(bufs.at[1, pl.ds(0, ZB)], agg.at[pl.ds(off, ZB)])
            return carry
        lax.fori_loop(0, nblk, zcopy, 0)
        plsc.subcore_barrier()

        pltpu.async_copy(x_hbm.at[colv.at[0]], bufs.at[0], gsem)
        pltpu.async_copy(bufs.at[1], agg.at[rowv.at[0, 0]], ssem, add=True)

        def chunk_body(i, issue=True):
            h = i % 2
            q = i % supc
            rrow = rowv.at[i // supc % 2, q]
            pltpu.make_async_copy(x_hbm.at[colv.at[i]], bufs.at[h],
                                  gsem).wait()
            pltpu.make_async_copy(bufs.at[1 - h], agg.at[rrow], ssem).wait()
            if issue:
                pltpu.async_copy(x_hbm.at[colv.at[i + 1]], bufs.at[1 - h],
                                 gsem)

            @plsc.parallel_loop(0, GW, unroll=2)
            def edge_body(ei):
                base = q * GW + ei
                base16 = base // 16 * 16
                grp = valv[pl.ds(base16, 16)]
                v16 = grp.at[jnp.full((16,), base - base16, jnp.int32)].get(
                    mode="promise_in_bounds")
                for u in range(d // 16):
                    sl = (h, ei, pl.ds(u * 16, 16))
                    bufs[sl] = bufs[sl] * v16
            pltpu.async_copy(bufs.at[h], agg.at[rrow], ssem, add=True)

        def seg(lo, hi):
            def sbody(i, carry):
                chunk_body(i)
                return carry
            lax.fori_loop(lo, hi, sbody, 0)

        def sup_body(k, carry):
            lo = k * supc
            seg(lo, lo + 2)
            pltpu.async_copy(
                row_hbm.at[pl.ds(pl.multiple_of(cb + (k + 1) * supc, 8),
                                 supc)],
                rowv.at[(k + 1) % 2], stsem)
            seg(lo + 2, lo + supc)
            pltpu.make_async_copy(row_hbm.at[pl.ds(cb, supc)],
                                  rowv.at[(k + 1) % 2], stsem).wait()
            pltpu.async_copy(val_hbm.at[pl.ds(ebase + (k + 1) * SUP, SUP)],
                             valv, stsem)
            pltpu.make_async_copy(val_hbm.at[pl.ds(ebase, SUP)], valv,
                                  stsem).wait()
            return carry
        lax.fori_loop(0, nsup - 1, sup_body, 0)

        lo = (nsup - 1) * supc
        seg(lo, lo + supc - 1)
        chunk_body(nch - 1, issue=False)

        pltpu.make_async_copy(bufs.at[0], agg.at[rowv.at[0, 0]], ssem).wait()
        plsc.subcore_barrier()

        def drain(t, carry):
            off = pl.multiple_of(sbase + t * ZB, 8)
            pltpu.sync_copy(
                agg.at[pl.ds(off, ZB)],
                out_hbm.at[pl.ds(pl.multiple_of(c * n + sbase + t * ZB, 8), ZB)],
            )
            return carry
        lax.fori_loop(0, nblk, drain, 0)

    return body(x2, col2, row2, vals)


def _mm_relu_kernel(a_ref, w_ref, o_ref):
    o_ref[...] = jnp.maximum(
        jnp.dot(a_ref[...], w_ref[...], preferred_element_type=jnp.float32),
        0.0,
    )


def kernel(x, adj_indices, adj_values, W):
    b, n, d = x.shape
    e = adj_indices.shape[1]
    dout = W.shape[1]

    row = adj_indices[..., 0].astype(jnp.int32)
    col = adj_indices[..., 1].astype(jnp.int32)
    colg = col + (jnp.arange(b, dtype=jnp.int32) * n)[:, None]
    col2 = colg.reshape(b * e // GW, GW)
    row2 = row.reshape(b * e // GW, GW)
    vals = adj_values.reshape(b * e)
    x2 = x.reshape(b * n, d)

    agg = _sc_aggregate(x2, col2, row2, vals, n=n, d=d, e=e)

    rows_total = b * n
    blk = 2000
    out = pl.pallas_call(
        _mm_relu_kernel,
        grid=(rows_total // blk,),
        in_specs=[
            pl.BlockSpec((blk, d), lambda i: (i, 0)),
            pl.BlockSpec((d, dout), lambda i: (0, 0)),
        ],
        out_specs=pl.BlockSpec((blk, dout), lambda i: (i, 0)),
        out_shape=jax.ShapeDtypeStruct((rows_total, dout), jnp.float32),
    )(agg, W)
    return out.reshape(b, n, dout)
